# baseline (speedup 1.0000x reference)
import functools

import jax
import jax.numpy as jnp
from jax import lax
from jax.experimental import pallas as pl
from jax.experimental.pallas import tpu as pltpu
from jax.experimental.pallas import tpu_sc as plsc

VOCAB_SIZE = 1000000
EMBEDDING_DIM = 32
BATCH = 16384
SEQ_LEN = 50

_NC = 2
_NS = 16
_NW = _NC * _NS

_BPT = BATCH // _NW
_NBUF = 4
_NT = 2
_LANES = 16


def _body(table_hbm, idxT_hbm, out_hbm, idx_v, rows_v, tbuf, gsems, osems):
    wid = lax.axis_index("s") * _NC + lax.axis_index("c")
    b0 = wid * _BPT

    def gather(slot, s):
        return pltpu.make_async_copy(table_hbm.at[idx_v.at[s]],
                                     rows_v.at[slot], gsems.at[slot])

    def outcopy(ts, s):
        return pltpu.make_async_copy(tbuf.at[ts],
                                     out_hbm.at[s, :, pl.ds(b0, _BPT)],
                                     osems.at[ts])

    bvecs = [v * _LANES + lax.iota(jnp.int32, _LANES)
             for v in range(_BPT // _LANES)]

    def transpose(slot, ts):
        rows = rows_v.at[slot]

        def percol(c, carry):
            cvec = jnp.full((_LANES,), 0, jnp.int32) + c
            for v in range(_BPT // _LANES):
                vals = plsc.load_gather(rows, [bvecs[v], cvec])
                tbuf[ts, c, pl.ds(v * _LANES, _LANES)] = vals
            return carry

        lax.fori_loop(0, EMBEDDING_DIM, percol, 0, unroll=4)

    pltpu.sync_copy(idxT_hbm.at[:, pl.ds(b0, _BPT)], idx_v)

    for s in range(_NBUF):
        gather(s, s).start()

    def group(g, carry):
        for k in range(_NBUF):
            s = g * _NBUF + k
            ts = k % _NT

            @pl.when(s < SEQ_LEN)
            def _():
                gather(k, s).wait()

                @pl.when(s >= _NT)
                def _():
                    outcopy(ts, s - _NT).wait()

                transpose(k, ts)
                outcopy(ts, s).start()

                @pl.when(s + _NBUF < SEQ_LEN)
                def _():
                    gather(k, s + _NBUF).start()

        return carry

    lax.fori_loop(0, (SEQ_LEN + _NBUF - 1) // _NBUF, group, 0)

    for s in (SEQ_LEN - 2, SEQ_LEN - 1):
        outcopy(s % _NT, s).wait()


@jax.jit
def _embedding_gather(idxT, table):
    mesh = plsc.VectorSubcoreMesh(core_axis_name="c", subcore_axis_name="s")
    run = pl.kernel(
        _body,
        out_type=jax.ShapeDtypeStruct((SEQ_LEN, EMBEDDING_DIM, BATCH),
                                      jnp.float32),
        mesh=mesh,
        scratch_types=[
            pltpu.VMEM((SEQ_LEN, _BPT), jnp.int32),
            pltpu.VMEM((_NBUF, _BPT, EMBEDDING_DIM), jnp.float32),
            pltpu.VMEM((_NT, EMBEDDING_DIM, _BPT), jnp.float32),
            pltpu.SemaphoreType.DMA((_NBUF,)),
            pltpu.SemaphoreType.DMA((_NT,)),
        ],
        compiler_params=pltpu.CompilerParams(use_tc_tiling_on_sc=False,
                                             needs_layout_passes=False),
    )
    return run(table, idxT)


def kernel(inputs, shared_weights):
    idxT = inputs.astype(jnp.int32).T
    out3 = _embedding_gather(idxT, shared_weights)
    return out3.transpose(2, 0, 1)


# speedup vs baseline: 1.0649x; 1.0649x over previous
import functools

import jax
import jax.numpy as jnp
from jax import lax
from jax.experimental import pallas as pl
from jax.experimental.pallas import tpu as pltpu
from jax.experimental.pallas import tpu_sc as plsc

VOCAB_SIZE = 1000000
EMBEDDING_DIM = 32
BATCH = 16384
SEQ_LEN = 50

_NC = 2
_NS = 16
_NW = _NC * _NS

_BPT = BATCH // _NW
_CH = 128
_CPS = _BPT // _CH
_STEPS = SEQ_LEN * _CPS
_NBUF = 4
_NT = 2
_LANES = 16
_VPC = _CH // _LANES


def _body(table_hbm, idxT_hbm, out_hbm, idx_v, gidx_v, rows_v, tbuf,
          gsems, osems):
    wid = lax.axis_index("s") * _NC + lax.axis_index("c")
    b0 = wid * _BPT

    def gather(slot):
        return pltpu.make_async_copy(table_hbm.at[gidx_v.at[slot]],
                                     rows_v.at[slot], gsems.at[slot])

    def outcopy(ts, step):
        s = lax.div(step, _CPS)
        ch = lax.rem(step, _CPS)
        return pltpu.make_async_copy(
            tbuf.at[ts],
            out_hbm.at[s, :, pl.ds(b0 + ch * _CH, _CH)],
            osems.at[ts])

    def prep_gidx(slot, step):
        s = lax.div(step, _CPS)
        ch = lax.rem(step, _CPS)
        for v in range(_VPC):
            iv = idx_v[s, pl.ds(ch * _CH + v * _LANES, _LANES)]
            gidx_v[slot, pl.ds(v * _LANES, _LANES)] = iv >> 2

    def transpose(slot, ts, step):
        rows = rows_v.at[slot]
        s = lax.div(step, _CPS)
        ch = lax.rem(step, _CPS)
        qoffs = []
        bvecs = []
        for v in range(_VPC):
            iv = idx_v[s, pl.ds(ch * _CH + v * _LANES, _LANES)]
            qoffs.append((iv & 3) << 5)
            bvecs.append(v * _LANES + lax.iota(jnp.int32, _LANES))

        for c in range(EMBEDDING_DIM):
            for v in range(_VPC):
                vals = plsc.load_gather(rows, [bvecs[v], qoffs[v] + c])
                tbuf[ts, c, pl.ds(v * _LANES, _LANES)] = vals

    pltpu.sync_copy(idxT_hbm.at[:, pl.ds(b0, _BPT)], idx_v)

    for step in range(_NBUF):
        prep_gidx(step, step)
        gather(step).start()

    def group(g, carry):
        for k in range(_NBUF):
            step = g * _NBUF + k
            ts = k % _NT
            gather(k).wait()

            @pl.when(step >= _NT)
            def _():
                outcopy(ts, step - _NT).wait()

            transpose(k, ts, step)
            outcopy(ts, step).start()

            @pl.when(step + _NBUF < _STEPS)
            def _():
                prep_gidx(k, step + _NBUF)
                gather(k).start()

        return carry

    lax.fori_loop(0, _STEPS // _NBUF, group, 0)

    for step in (_STEPS - 2, _STEPS - 1):
        outcopy(step % _NT, step).wait()


@jax.jit
def _embedding_gather(idxT, table128):
    mesh = plsc.VectorSubcoreMesh(core_axis_name="c", subcore_axis_name="s")
    run = pl.kernel(
        _body,
        out_type=jax.ShapeDtypeStruct((SEQ_LEN, EMBEDDING_DIM, BATCH),
                                      jnp.float32),
        mesh=mesh,
        scratch_types=[
            pltpu.VMEM((SEQ_LEN, _BPT), jnp.int32),
            pltpu.VMEM((_NBUF, _CH), jnp.int32),
            pltpu.VMEM((_NBUF, _CH, 4 * EMBEDDING_DIM), jnp.float32),
            pltpu.VMEM((_NT, EMBEDDING_DIM, _CH), jnp.float32),
            pltpu.SemaphoreType.DMA((_NBUF,)),
            pltpu.SemaphoreType.DMA((_NT,)),
        ],
        compiler_params=pltpu.CompilerParams(use_tc_tiling_on_sc=True,
                                             needs_layout_passes=False),
    )
    return run(table128, idxT)


def kernel(inputs, shared_weights):
    idxT = inputs.astype(jnp.int32).T
    table128 = shared_weights.reshape(VOCAB_SIZE // 4, 4 * EMBEDDING_DIM)
    out3 = _embedding_gather(idxT, table128)
    return out3.transpose(2, 0, 1)
